# baseline (speedup 1.0000x reference)
import jax
import jax.numpy as jnp
from jax.experimental import pallas as pl
from jax.experimental.pallas import tpu as pltpu

_B, _S, _D, _DFF, _E, _K = 2, 4096, 1024, 2048, 8, 4
_TBLK = 1024
_FBLK = 512
_RBLK = 1024


def _routing_kernel(x_ref, gate_w_ref, gate_b_ref, idx_ref, acc_ref):
    i = pl.program_id(0)
    n = pl.num_programs(0)

    @pl.when(i == 0)
    def _():
        acc_ref[...] = jnp.zeros_like(acc_ref)

    acc_ref[...] += jnp.sum(x_ref[...], axis=0, keepdims=True)

    @pl.when(i == n - 1)
    def _():
        ksum = acc_ref[...]
        gw0 = jax.lax.dot_general(
            ksum, gate_w_ref[...], (((1,), (1,)), ((), ())),
            preferred_element_type=jnp.float32,
            precision=jax.lax.Precision.HIGHEST)
        gw0 = gw0 + gate_b_ref[...]
        iota = jax.lax.broadcasted_iota(jnp.int32, (1, _E), 1)
        vals = gw0
        for j in range(_K):
            m = jnp.max(vals)
            lane = jnp.min(jnp.where(vals == m, iota, _E))
            idx_ref[j] = lane.astype(jnp.int32)
            vals = jnp.where(iota == lane, -jnp.inf, vals)


def _moe_kernel(idx_ref, x_ref, gate_w_ref, gate_b_ref, w1_ref, b1_ref,
                w2_ref, b2_ref, out_ref, g_scr, xb_scr):
    k = pl.program_id(1)
    f = pl.program_id(2)
    first = jnp.logical_and(k == 0, f == 0)

    @pl.when(first)
    def _():
        xf = x_ref[...]
        lg8 = jax.lax.dot_general(
            xf, gate_w_ref[...], (((1,), (1,)), ((), ())),
            preferred_element_type=jnp.float32,
            precision=jax.lax.Precision.HIGHEST)
        lg8 = lg8 + gate_b_ref[...]
        rows = jax.lax.broadcasted_iota(jnp.int32, (_E, _K), 0)
        idx_mat = jnp.concatenate(
            [jnp.full((_E, 1), idx_ref[j], dtype=jnp.int32) for j in range(_K)],
            axis=1)
        onehot = (rows == idx_mat).astype(jnp.float32)
        l4 = jax.lax.dot_general(
            lg8, onehot, (((1,), (0,)), ((), ())),
            preferred_element_type=jnp.float32,
            precision=jax.lax.Precision.HIGHEST)
        m = jnp.max(l4, axis=1, keepdims=True)
        ex = jnp.exp(l4 - m)
        g_scr[...] = ex / jnp.sum(ex, axis=1, keepdims=True)
        xb_scr[...] = xf.astype(jnp.bfloat16)
        out_ref[...] = jnp.zeros_like(out_ref)

    xb = xb_scr[...]
    w1b = w1_ref[0].astype(jnp.bfloat16)
    h = jnp.dot(xb, w1b, preferred_element_type=jnp.float32)
    h = jnp.maximum(h + b1_ref[0], 0.0)
    hb = h.astype(jnp.bfloat16)
    w2b = w2_ref[0].astype(jnp.bfloat16)
    y = jnp.dot(hb, w2b, preferred_element_type=jnp.float32)
    y = y + (f == 0).astype(jnp.float32) * b2_ref[0]
    kmask = (jax.lax.broadcasted_iota(jnp.int32, (1, _K), 1) == k)
    g_k = jnp.sum(g_scr[...] * kmask.astype(jnp.float32), axis=1, keepdims=True)
    out_ref[...] += g_k * y


def kernel(x, gate_w, gate_b, w1, b1, w2, b2):
    bs = _B * _S
    x2d = x.reshape(bs, _D)
    gate_b2d = gate_b.reshape(1, _E)
    b1r = b1.reshape(_E, 1, _DFF)
    b2r = b2.reshape(_E, 1, _D)

    idx = pl.pallas_call(
        _routing_kernel,
        grid=(bs // _RBLK,),
        in_specs=[
            pl.BlockSpec((_RBLK, _D), lambda i: (i, 0)),
            pl.BlockSpec((_E, _D), lambda i: (0, 0)),
            pl.BlockSpec((1, _E), lambda i: (0, 0)),
        ],
        out_specs=pl.BlockSpec(memory_space=pltpu.SMEM),
        out_shape=jax.ShapeDtypeStruct((_K,), jnp.int32),
        scratch_shapes=[pltpu.VMEM((1, _D), jnp.float32)],
        compiler_params=pltpu.CompilerParams(
            dimension_semantics=("arbitrary",)),
    )(x2d, gate_w, gate_b2d)

    grid = (bs // _TBLK, _K, _DFF // _FBLK)
    out2d = pl.pallas_call(
        _moe_kernel,
        grid_spec=pltpu.PrefetchScalarGridSpec(
            num_scalar_prefetch=1,
            grid=grid,
            in_specs=[
                pl.BlockSpec((_TBLK, _D), lambda t, k, f, idx: (t, 0)),
                pl.BlockSpec((_E, _D), lambda t, k, f, idx: (0, 0)),
                pl.BlockSpec((1, _E), lambda t, k, f, idx: (0, 0)),
                pl.BlockSpec((1, _D, _FBLK), lambda t, k, f, idx: (idx[k], 0, f)),
                pl.BlockSpec((1, 1, _FBLK), lambda t, k, f, idx: (idx[k], 0, f)),
                pl.BlockSpec((1, _FBLK, _D), lambda t, k, f, idx: (idx[k], f, 0)),
                pl.BlockSpec((1, 1, _D), lambda t, k, f, idx: (idx[k], 0, 0)),
            ],
            out_specs=pl.BlockSpec((_TBLK, _D), lambda t, k, f, idx: (t, 0)),
            scratch_shapes=[
                pltpu.VMEM((_TBLK, _K), jnp.float32),
                pltpu.VMEM((_TBLK, _D), jnp.bfloat16),
            ],
        ),
        out_shape=jax.ShapeDtypeStruct((bs, _D), jnp.float32),
        compiler_params=pltpu.CompilerParams(
            dimension_semantics=("parallel", "arbitrary", "arbitrary"),
            vmem_limit_bytes=64 * 1024 * 1024),
    )(idx, x2d, gate_w, gate_b2d, w1, b1r, w2, b2r)

    return out2d.reshape(_B, _S, _D)


# speedup vs baseline: 1.0610x; 1.0610x over previous
import jax
import jax.numpy as jnp
from jax.experimental import pallas as pl
from jax.experimental.pallas import tpu as pltpu

_B, _S, _D, _DFF, _E, _K = 2, 4096, 1024, 2048, 8, 4
_TBLK = 2048
_FBLK = 512
_RBLK = 1024


def _routing_kernel(x_ref, gate_w_ref, gate_b_ref, idx_ref, acc_ref):
    i = pl.program_id(0)
    n = pl.num_programs(0)

    @pl.when(i == 0)
    def _():
        acc_ref[...] = jnp.zeros_like(acc_ref)

    acc_ref[...] += jnp.sum(x_ref[...], axis=0, keepdims=True)

    @pl.when(i == n - 1)
    def _():
        ksum = acc_ref[...]
        gw0 = jax.lax.dot_general(
            ksum, gate_w_ref[...], (((1,), (1,)), ((), ())),
            preferred_element_type=jnp.float32,
            precision=jax.lax.Precision.HIGHEST)
        gw0 = gw0 + gate_b_ref[...]
        iota = jax.lax.broadcasted_iota(jnp.int32, (1, _E), 1)
        vals = gw0
        for j in range(_K):
            m = jnp.max(vals)
            lane = jnp.min(jnp.where(vals == m, iota, _E))
            idx_ref[j] = lane.astype(jnp.int32)
            vals = jnp.where(iota == lane, -jnp.inf, vals)


def _moe_kernel(idx_ref, x_ref, gate_w_ref, gate_b_ref, w1_ref, b1_ref,
                w2_ref, b2_ref, out_ref, g_scr, xb_scr):
    k = pl.program_id(1)
    f = pl.program_id(2)
    first = jnp.logical_and(k == 0, f == 0)

    @pl.when(first)
    def _():
        xf = x_ref[...]
        lg8 = jax.lax.dot_general(
            xf, gate_w_ref[...], (((1,), (1,)), ((), ())),
            preferred_element_type=jnp.float32,
            precision=jax.lax.Precision.HIGHEST)
        lg8 = lg8 + gate_b_ref[...]
        rows = jax.lax.broadcasted_iota(jnp.int32, (_E, _K), 0)
        idx_mat = jnp.concatenate(
            [jnp.full((_E, 1), idx_ref[j], dtype=jnp.int32) for j in range(_K)],
            axis=1)
        onehot = (rows == idx_mat).astype(jnp.float32)
        l4 = jax.lax.dot_general(
            lg8, onehot, (((1,), (0,)), ((), ())),
            preferred_element_type=jnp.float32,
            precision=jax.lax.Precision.HIGHEST)
        m = jnp.max(l4, axis=1, keepdims=True)
        ex = jnp.exp(l4 - m)
        g_scr[...] = ex / jnp.sum(ex, axis=1, keepdims=True)
        xb_scr[...] = xf.astype(jnp.bfloat16)
        out_ref[...] = jnp.zeros_like(out_ref)

    xb = xb_scr[...]
    w1b = w1_ref[0].astype(jnp.bfloat16)
    h = jnp.dot(xb, w1b, preferred_element_type=jnp.float32)
    h = jnp.maximum(h + b1_ref[0], 0.0)
    hb = h.astype(jnp.bfloat16)
    w2b = w2_ref[0].astype(jnp.bfloat16)
    y = jnp.dot(hb, w2b, preferred_element_type=jnp.float32)
    y = y + (f == 0).astype(jnp.float32) * b2_ref[0]
    kmask = (jax.lax.broadcasted_iota(jnp.int32, (1, _K), 1) == k)
    g_k = jnp.sum(g_scr[...] * kmask.astype(jnp.float32), axis=1, keepdims=True)
    out_ref[...] += g_k * y


def kernel(x, gate_w, gate_b, w1, b1, w2, b2):
    bs = _B * _S
    x2d = x.reshape(bs, _D)
    gate_b2d = gate_b.reshape(1, _E)
    b1r = b1.reshape(_E, 1, _DFF)
    b2r = b2.reshape(_E, 1, _D)

    idx = pl.pallas_call(
        _routing_kernel,
        grid=(bs // _RBLK,),
        in_specs=[
            pl.BlockSpec((_RBLK, _D), lambda i: (i, 0)),
            pl.BlockSpec((_E, _D), lambda i: (0, 0)),
            pl.BlockSpec((1, _E), lambda i: (0, 0)),
        ],
        out_specs=pl.BlockSpec(memory_space=pltpu.SMEM),
        out_shape=jax.ShapeDtypeStruct((_K,), jnp.int32),
        scratch_shapes=[pltpu.VMEM((1, _D), jnp.float32)],
        compiler_params=pltpu.CompilerParams(
            dimension_semantics=("arbitrary",)),
    )(x2d, gate_w, gate_b2d)

    grid = (bs // _TBLK, _K, _DFF // _FBLK)
    out2d = pl.pallas_call(
        _moe_kernel,
        grid_spec=pltpu.PrefetchScalarGridSpec(
            num_scalar_prefetch=1,
            grid=grid,
            in_specs=[
                pl.BlockSpec((_TBLK, _D), lambda t, k, f, idx: (t, 0)),
                pl.BlockSpec((_E, _D), lambda t, k, f, idx: (0, 0)),
                pl.BlockSpec((1, _E), lambda t, k, f, idx: (0, 0)),
                pl.BlockSpec((1, _D, _FBLK), lambda t, k, f, idx: (idx[k], 0, f)),
                pl.BlockSpec((1, 1, _FBLK), lambda t, k, f, idx: (idx[k], 0, f)),
                pl.BlockSpec((1, _FBLK, _D), lambda t, k, f, idx: (idx[k], f, 0)),
                pl.BlockSpec((1, 1, _D), lambda t, k, f, idx: (idx[k], 0, 0)),
            ],
            out_specs=pl.BlockSpec((_TBLK, _D), lambda t, k, f, idx: (t, 0)),
            scratch_shapes=[
                pltpu.VMEM((_TBLK, _K), jnp.float32),
                pltpu.VMEM((_TBLK, _D), jnp.bfloat16),
            ],
        ),
        out_shape=jax.ShapeDtypeStruct((bs, _D), jnp.float32),
        compiler_params=pltpu.CompilerParams(
            dimension_semantics=("parallel", "arbitrary", "arbitrary"),
            vmem_limit_bytes=64 * 1024 * 1024),
    )(idx, x2d, gate_w, gate_b2d, w1, b1r, w2, b2r)

    return out2d.reshape(_B, _S, _D)


# speedup vs baseline: 1.1313x; 1.0663x over previous
import jax
import jax.numpy as jnp
from jax.experimental import pallas as pl
from jax.experimental.pallas import tpu as pltpu

_B, _S, _D, _DFF, _E, _K = 2, 4096, 1024, 2048, 8, 4
_TBLK = 2048
_FBLK = 512
_RBLK = 1024


def _prep_kernel(x_ref, gate_w_ref, gate_b_ref, idx_ref, xb_ref, lg8_ref,
                 acc_ref):
    i = pl.program_id(0)
    n = pl.num_programs(0)

    @pl.when(i == 0)
    def _():
        acc_ref[...] = jnp.zeros_like(acc_ref)

    xf = x_ref[...]
    acc_ref[...] += jnp.sum(xf, axis=0, keepdims=True)
    xb = xf.astype(jnp.bfloat16)
    xb_ref[...] = xb
    lg8_ref[...] = jax.lax.dot_general(
        xb, gate_w_ref[...].astype(jnp.bfloat16), (((1,), (1,)), ((), ())),
        preferred_element_type=jnp.float32) + gate_b_ref[...]

    @pl.when(i == n - 1)
    def _():
        ksum = acc_ref[...]
        gw0 = jax.lax.dot_general(
            ksum, gate_w_ref[...], (((1,), (1,)), ((), ())),
            preferred_element_type=jnp.float32,
            precision=jax.lax.Precision.HIGHEST)
        gw0 = gw0 + gate_b_ref[...]
        iota = jax.lax.broadcasted_iota(jnp.int32, (1, _E), 1)
        vals = gw0
        for j in range(_K):
            m = jnp.max(vals)
            lane = jnp.min(jnp.where(vals == m, iota, _E))
            idx_ref[j] = lane.astype(jnp.int32)
            vals = jnp.where(iota == lane, -jnp.inf, vals)


def _gates_kernel(idx_ref, lg8_ref, g_ref):
    rows = jax.lax.broadcasted_iota(jnp.int32, (_E, _K), 0)
    idx_mat = jnp.concatenate(
        [jnp.full((_E, 1), idx_ref[j], dtype=jnp.int32) for j in range(_K)],
        axis=1)
    onehot = (rows == idx_mat).astype(jnp.float32)
    l4 = jnp.dot(lg8_ref[...], onehot,
                 preferred_element_type=jnp.float32)
    m = jnp.max(l4, axis=1, keepdims=True)
    ex = jnp.exp(l4 - m)
    g_ref[...] = ex / jnp.sum(ex, axis=1, keepdims=True)


def _moe_kernel(idx_ref, xb_ref, g_ref, w1_ref, b1_ref, w2_ref, b2_ref,
                out_ref):
    k = pl.program_id(1)
    f = pl.program_id(2)

    @pl.when(jnp.logical_and(k == 0, f == 0))
    def _():
        out_ref[...] = jnp.zeros_like(out_ref)

    h = jnp.dot(xb_ref[...], w1_ref[0].astype(jnp.bfloat16),
                preferred_element_type=jnp.float32)
    h = jnp.maximum(h + b1_ref[0], 0.0).astype(jnp.bfloat16)
    y = jnp.dot(h, w2_ref[0].astype(jnp.bfloat16),
                preferred_element_type=jnp.float32)
    y = y + (f == 0).astype(jnp.float32) * b2_ref[0]
    kmask = (jax.lax.broadcasted_iota(jnp.int32, (1, _K), 1) == k)
    g_k = jnp.sum(g_ref[...] * kmask.astype(jnp.float32), axis=1,
                  keepdims=True)
    out_ref[...] += g_k * y


def kernel(x, gate_w, gate_b, w1, b1, w2, b2):
    bs = _B * _S
    x2d = x.reshape(bs, _D)
    gate_b2d = gate_b.reshape(1, _E)
    b1r = b1.reshape(_E, 1, _DFF)
    b2r = b2.reshape(_E, 1, _D)

    idx, xb, lg8 = pl.pallas_call(
        _prep_kernel,
        grid=(bs // _RBLK,),
        in_specs=[
            pl.BlockSpec((_RBLK, _D), lambda i: (i, 0)),
            pl.BlockSpec((_E, _D), lambda i: (0, 0)),
            pl.BlockSpec((1, _E), lambda i: (0, 0)),
        ],
        out_specs=[
            pl.BlockSpec(memory_space=pltpu.SMEM),
            pl.BlockSpec((_RBLK, _D), lambda i: (i, 0)),
            pl.BlockSpec((_RBLK, _E), lambda i: (i, 0)),
        ],
        out_shape=[
            jax.ShapeDtypeStruct((_K,), jnp.int32),
            jax.ShapeDtypeStruct((bs, _D), jnp.bfloat16),
            jax.ShapeDtypeStruct((bs, _E), jnp.float32),
        ],
        scratch_shapes=[pltpu.VMEM((1, _D), jnp.float32)],
        compiler_params=pltpu.CompilerParams(
            dimension_semantics=("arbitrary",)),
    )(x2d, gate_w, gate_b2d)

    g = pl.pallas_call(
        _gates_kernel,
        grid_spec=pltpu.PrefetchScalarGridSpec(
            num_scalar_prefetch=1,
            grid=(1,),
            in_specs=[pl.BlockSpec((bs, _E), lambda i, idx: (0, 0))],
            out_specs=pl.BlockSpec((bs, _K), lambda i, idx: (0, 0)),
        ),
        out_shape=jax.ShapeDtypeStruct((bs, _K), jnp.float32),
    )(idx, lg8)

    grid = (bs // _TBLK, _K, _DFF // _FBLK)
    out2d = pl.pallas_call(
        _moe_kernel,
        grid_spec=pltpu.PrefetchScalarGridSpec(
            num_scalar_prefetch=1,
            grid=grid,
            in_specs=[
                pl.BlockSpec((_TBLK, _D), lambda t, k, f, idx: (t, 0)),
                pl.BlockSpec((_TBLK, _K), lambda t, k, f, idx: (t, 0)),
                pl.BlockSpec((1, _D, _FBLK), lambda t, k, f, idx: (idx[k], 0, f)),
                pl.BlockSpec((1, 1, _FBLK), lambda t, k, f, idx: (idx[k], 0, f)),
                pl.BlockSpec((1, _FBLK, _D), lambda t, k, f, idx: (idx[k], f, 0)),
                pl.BlockSpec((1, 1, _D), lambda t, k, f, idx: (idx[k], 0, 0)),
            ],
            out_specs=pl.BlockSpec((_TBLK, _D), lambda t, k, f, idx: (t, 0)),
        ),
        out_shape=jax.ShapeDtypeStruct((bs, _D), jnp.float32),
        compiler_params=pltpu.CompilerParams(
            dimension_semantics=("parallel", "arbitrary", "arbitrary"),
            vmem_limit_bytes=64 * 1024 * 1024),
    )(idx, xb, g, w1, b1r, w2, b2r)

    return out2d.reshape(_B, _S, _D)
